# baseline (speedup 1.0000x reference)
import functools

import jax
import jax.numpy as jnp
from jax import lax
from jax.experimental import pallas as pl
from jax.experimental.pallas import tpu as pltpu
from jax.experimental.pallas import tpu_sc as plsc

_L = 16
_H = 128


@functools.lru_cache(maxsize=None)
def _build(batch, embed_dim):
    info = plsc.get_sparse_core_info()
    nw = info.num_cores * info.num_subcores
    bpw = batch // nw
    npass = bpw // _H
    nchunk = embed_dim // _L
    mesh = plsc.VectorSubcoreMesh(core_axis_name="c", subcore_axis_name="s")

    @functools.partial(
        pl.kernel,
        mesh=mesh,
        out_type=jax.ShapeDtypeStruct((batch,), jnp.float32),
        compiler_params=pltpu.CompilerParams(needs_layout_passes=False),
        scratch_types=[
            pltpu.VMEM((bpw,), jnp.int32),
            pltpu.VMEM((bpw,), jnp.int32),
            pltpu.VMEM((_H, 64), jnp.float32),
            pltpu.VMEM((_H, 64), jnp.float32),
            pltpu.VMEM((_H, 1), jnp.float32),
            pltpu.VMEM((_H, 1), jnp.float32),
            pltpu.VMEM((bpw,), jnp.float32),
            pltpu.VMEM((_L,), jnp.float32),
            pltpu.SemaphoreType.DMA,
            pltpu.SemaphoreType.DMA,
            pltpu.SemaphoreType.DMA,
            pltpu.SemaphoreType.DMA,
        ],
    )
    def mf(src_hbm, dst_hbm, uemb_hbm, ubias_hbm, iemb_hbm, ibias_hbm,
           mean_hbm, out_hbm, sidx, didx, urows, irows, ubc, ibc,
           outv, meanv, sem, sem2, sem3, sem4):
        wid = lax.axis_index("s") * info.num_cores + lax.axis_index("c")
        base = wid * bpw

        pltpu.sync_copy(src_hbm.at[wid], sidx)
        pltpu.sync_copy(dst_hbm.at[wid], didx)
        pltpu.sync_copy(mean_hbm, meanv)

        lanes = lax.iota(jnp.int32, _L)
        zvec = jnp.zeros((_L,), jnp.int32)
        mean_vec = meanv[...]

        for p in range(npass):
            def fire(g, _):
                su = sidx[pl.ds(p * _H + g * _L, _L)]
                sv = didx[pl.ds(p * _H + g * _L, _L)]
                for j in range(_L):
                    i = g * _L + j
                    ru = su[j] * 0
                    rv = sv[j] * 0
                    pltpu.async_copy(uemb_hbm.at[pl.ds(ru, 1), :],
                                     urows.at[pl.ds(i, 1), :], sem)
                    pltpu.async_copy(iemb_hbm.at[pl.ds(rv, 1), :],
                                     irows.at[pl.ds(i, 1), :], sem2)
                return 0

            lax.fori_loop(0, _H // _L, fire, 0)

            pltpu.make_async_copy(uemb_hbm.at[pl.ds(0, _H), :], urows, sem).wait()
            pltpu.make_async_copy(iemb_hbm.at[pl.ds(0, _H), :], irows, sem2).wait()

            def group(g, _):
                out_vec = jnp.zeros((_L,), jnp.float32)
                for j in range(_L):
                    r = g * _L + j
                    acc = (urows[r, pl.ds(0, _L)] * irows[r, pl.ds(0, _L)])
                    for c in range(1, nchunk):
                        acc = acc + (urows[r, pl.ds(c * _L, _L)]
                                     * irows[r, pl.ds(c * _L, _L)])
                    s = jnp.sum(acc)
                    out_vec = jnp.where(lanes == j, s, out_vec)
                rows16 = lanes + g * _L
                bu = plsc.load_gather(ubc, [rows16, zvec])
                bi = plsc.load_gather(ibc, [rows16, zvec])
                outv[pl.ds(p * _H + g * _L, _L)] = (
                    out_vec + bu + bi + mean_vec)
                return 0

            lax.fori_loop(0, _H // _L, group, 0)

        pltpu.sync_copy(outv, out_hbm.at[pl.ds(base, bpw)])

    return mf, nw, bpw


def kernel(src, dst, user_emb, user_bias, item_emb, item_bias, mean):
    batch = src.shape[0]
    embed_dim = user_emb.shape[1]
    mf, nw, bpw = _build(batch, embed_dim)
    src2 = src.astype(jnp.int32).reshape(nw, bpw)
    dst2 = dst.astype(jnp.int32).reshape(nw, bpw)
    mean16 = jnp.broadcast_to(mean.reshape(()), (_L,)).astype(jnp.float32)
    return mf(src2, dst2, user_emb, user_bias, item_emb, item_bias, mean16)


# speedup vs baseline: 1.5239x; 1.5239x over previous
import functools

import jax
import jax.numpy as jnp
from jax import lax
from jax.experimental import pallas as pl
from jax.experimental.pallas import tpu as pltpu
from jax.experimental.pallas import tpu_sc as plsc

_L = 16
_H = 128


@functools.lru_cache(maxsize=None)
def _build(batch, embed_dim):
    info = plsc.get_sparse_core_info()
    nw = info.num_cores * info.num_subcores
    bpw = batch // nw
    npass = bpw // _H
    nchunk = embed_dim // _L
    mesh = plsc.VectorSubcoreMesh(core_axis_name="c", subcore_axis_name="s")

    @functools.partial(
        pl.kernel,
        mesh=mesh,
        out_type=jax.ShapeDtypeStruct((batch,), jnp.float32),
        compiler_params=pltpu.CompilerParams(needs_layout_passes=False),
        scratch_types=[
            pltpu.VMEM((bpw,), jnp.int32),
            pltpu.VMEM((bpw,), jnp.int32),
            pltpu.VMEM((_H, 64), jnp.float32),
            pltpu.VMEM((_H, 64), jnp.float32),
            pltpu.VMEM((_H, 1), jnp.float32),
            pltpu.VMEM((_H, 1), jnp.float32),
            pltpu.VMEM((bpw,), jnp.float32),
            pltpu.VMEM((_L,), jnp.float32),
            pltpu.SemaphoreType.DMA,
            pltpu.SemaphoreType.DMA,
            pltpu.SemaphoreType.DMA,
            pltpu.SemaphoreType.DMA,
        ],
    )
    def mf(src_hbm, dst_hbm, uemb_hbm, ubias_hbm, iemb_hbm, ibias_hbm,
           mean_hbm, out_hbm, sidx, didx, urows, irows, ubc, ibc,
           outv, meanv, sem, sem2, sem3, sem4):
        wid = lax.axis_index("s") * info.num_cores + lax.axis_index("c")
        base = wid * bpw

        pltpu.sync_copy(src_hbm.at[wid], sidx)
        pltpu.sync_copy(dst_hbm.at[wid], didx)
        pltpu.sync_copy(mean_hbm, meanv)

        lanes = lax.iota(jnp.int32, _L)
        zvec = jnp.zeros((_L,), jnp.int32)
        mean_vec = meanv[...]

        for p in range(npass):
            def fire(g, _):
                su = sidx[pl.ds(p * _H + g * _L, _L)]
                sv = didx[pl.ds(p * _H + g * _L, _L)]
                for j in range(_L):
                    i = g * _L + j
                    ru = su[j]
                    rv = sv[j]
                    pltpu.async_copy(uemb_hbm.at[pl.ds(ru, 1), :],
                                     urows.at[pl.ds(i, 1), :], sem)
                    pltpu.async_copy(iemb_hbm.at[pl.ds(rv, 1), :],
                                     irows.at[pl.ds(i, 1), :], sem2)
                    pltpu.async_copy(ubias_hbm.at[pl.ds(ru, 1), :],
                                     ubc.at[pl.ds(i, 1), :], sem3)
                    pltpu.async_copy(ibias_hbm.at[pl.ds(rv, 1), :],
                                     ibc.at[pl.ds(i, 1), :], sem4)
                return 0

            lax.fori_loop(0, _H // _L, fire, 0, unroll=4)

            pltpu.make_async_copy(uemb_hbm.at[pl.ds(0, _H), :], urows, sem).wait()
            pltpu.make_async_copy(iemb_hbm.at[pl.ds(0, _H), :], irows, sem2).wait()
            pltpu.make_async_copy(ubias_hbm.at[pl.ds(0, _H), :], ubc, sem3).wait()
            pltpu.make_async_copy(ibias_hbm.at[pl.ds(0, _H), :], ibc, sem4).wait()

            def group(g, _):
                out_vec = jnp.zeros((_L,), jnp.float32)
                for j in range(_L):
                    r = g * _L + j
                    acc = (urows[r, pl.ds(0, _L)] * irows[r, pl.ds(0, _L)])
                    for c in range(1, nchunk):
                        acc = acc + (urows[r, pl.ds(c * _L, _L)]
                                     * irows[r, pl.ds(c * _L, _L)])
                    s = jnp.sum(acc)
                    out_vec = jnp.where(lanes == j, s, out_vec)
                rows16 = lanes + g * _L
                bu = plsc.load_gather(ubc, [rows16, zvec])
                bi = plsc.load_gather(ibc, [rows16, zvec])
                outv[pl.ds(p * _H + g * _L, _L)] = (
                    out_vec + bu + bi + mean_vec)
                return 0

            lax.fori_loop(0, _H // _L, group, 0)

        pltpu.sync_copy(outv, out_hbm.at[pl.ds(base, bpw)])

    return mf, nw, bpw


def kernel(src, dst, user_emb, user_bias, item_emb, item_bias, mean):
    batch = src.shape[0]
    embed_dim = user_emb.shape[1]
    mf, nw, bpw = _build(batch, embed_dim)
    src2 = src.astype(jnp.int32).reshape(nw, bpw)
    dst2 = dst.astype(jnp.int32).reshape(nw, bpw)
    mean16 = jnp.broadcast_to(mean.reshape(()), (_L,)).astype(jnp.float32)
    return mf(src2, dst2, user_emb, user_bias, item_emb, item_bias, mean16)


# speedup vs baseline: 1.5710x; 1.0309x over previous
import functools

import jax
import jax.numpy as jnp
from jax import lax
from jax.experimental import pallas as pl
from jax.experimental.pallas import tpu as pltpu
from jax.experimental.pallas import tpu_sc as plsc

_L = 16
_CHUNK = 128


@functools.lru_cache(maxsize=None)
def _build(batch, embed_dim):
    info = plsc.get_sparse_core_info()
    nw = info.num_cores * info.num_subcores
    bpw = batch // nw
    nch = bpw // _CHUNK
    mesh = plsc.VectorSubcoreMesh(core_axis_name="c", subcore_axis_name="s")

    @functools.partial(
        pl.kernel,
        mesh=mesh,
        out_type=jax.ShapeDtypeStruct((batch,), jnp.float32),
        compiler_params=pltpu.CompilerParams(needs_layout_passes=False,
                                             use_tc_tiling_on_sc=False),
        scratch_types=[
            pltpu.VMEM((nch, _CHUNK), jnp.int32),
            pltpu.VMEM((nch, _CHUNK), jnp.int32),
            pltpu.VMEM((bpw, embed_dim), jnp.float32),
            pltpu.VMEM((bpw, embed_dim), jnp.float32),
            pltpu.VMEM((bpw,), jnp.float32),
            pltpu.VMEM((bpw,), jnp.float32),
            pltpu.VMEM((bpw,), jnp.float32),
            pltpu.VMEM((_L,), jnp.float32),
            pltpu.SemaphoreType.DMA,
        ],
    )
    def mf(src_hbm, dst_hbm, uemb_hbm, ubias_hbm, iemb_hbm, ibias_hbm,
           mean_hbm, out_hbm, sidx, didx, urows, irows, ub, ib, outv,
           meanv, sem):
        wid = lax.axis_index("s") * info.num_cores + lax.axis_index("c")
        base = wid * bpw

        pltpu.sync_copy(src_hbm.at[wid], sidx)
        pltpu.sync_copy(dst_hbm.at[wid], didx)
        pltpu.sync_copy(mean_hbm, meanv)

        descs = []
        for j in range(nch):
            rows = pl.ds(j * _CHUNK, _CHUNK)
            descs.append(pltpu.async_copy(uemb_hbm.at[sidx.at[j]],
                                          urows.at[rows, :], sem))
            descs.append(pltpu.async_copy(iemb_hbm.at[didx.at[j]],
                                          irows.at[rows, :], sem))
            descs.append(pltpu.async_copy(ubias_hbm.at[sidx.at[j]],
                                          ub.at[rows], sem))
            descs.append(pltpu.async_copy(ibias_hbm.at[didx.at[j]],
                                          ib.at[rows], sem))
        for d in descs:
            d.wait()

        mean_vec = meanv[...]
        lanes = lax.iota(jnp.int32, _L)

        def group_body(g, _):
            sl = pl.ds(g * _L, _L)
            out_vec = jnp.zeros((_L,), jnp.float32)
            for j in range(_L):
                r = g * _L + j
                acc = urows[r, pl.ds(0, _L)] * irows[r, pl.ds(0, _L)]
                for c in range(1, embed_dim // _L):
                    acc = acc + (urows[r, pl.ds(c * _L, _L)]
                                 * irows[r, pl.ds(c * _L, _L)])
                s = jnp.sum(acc)
                out_vec = jnp.where(lanes == j, s, out_vec)
            outv[sl] = out_vec + ub[sl] + ib[sl] + mean_vec
            return 0

        lax.fori_loop(0, bpw // _L, group_body, 0)

        pltpu.sync_copy(outv, out_hbm.at[pl.ds(base, bpw)])

    return mf, nw, nch


def kernel(src, dst, user_emb, user_bias, item_emb, item_bias, mean):
    batch = src.shape[0]
    embed_dim = user_emb.shape[1]
    mf, nw, nch = _build(batch, embed_dim)
    src3 = src.astype(jnp.int32).reshape(nw, nch, _CHUNK)
    dst3 = dst.astype(jnp.int32).reshape(nw, nch, _CHUNK)
    mean16 = jnp.broadcast_to(mean.reshape(()), (_L,)).astype(jnp.float32)
    return mf(src3, dst3, user_emb, user_bias.reshape(-1),
              item_emb, item_bias.reshape(-1), mean16)
